# Initial kernel scaffold; baseline (speedup 1.0000x reference)
import jax
import jax.numpy as jnp
from jax.experimental import pallas as pl


def kernel(hidden_states, router_logits, w13_weight, w2_weight):
    raise NotImplementedError("write your pallas kernel here")


# speedup vs baseline: 1.0348x; 1.0348x over previous
import functools

import jax
import jax.numpy as jnp
from jax.experimental import pallas as pl
from jax.experimental.pallas import tpu as pltpu

NUM_EXPERTS = 8
TOP_K = 2
HIDDEN = 1024
INTER = 2048
T = 2048

TB = 1024
FT = 512
NFT = INTER // FT


def _routing_cols(logits, e):
    iota = jax.lax.broadcasted_iota(jnp.int32, logits.shape, 1)
    m0 = jnp.max(logits, axis=1, keepdims=True)
    idx0 = jnp.min(jnp.where(logits == m0, iota, NUM_EXPERTS), axis=1,
                   keepdims=True)
    masked = jnp.where(iota == idx0, -jnp.inf, logits)
    m1 = jnp.max(masked, axis=1, keepdims=True)
    idx1 = jnp.min(jnp.where(masked == m1, iota, NUM_EXPERTS), axis=1,
                   keepdims=True)
    w0 = 1.0 / (1.0 + jnp.exp(m1 - m0))
    col = jnp.where(idx0 == e, w0, 0.0) + jnp.where(idx1 == e, 1.0 - w0, 0.0)
    return col


def _moe_body(x_ref, logits_ref, w13g_ref, w13u_ref, w2_ref, out_ref):
    e = pl.program_id(1)
    f = pl.program_id(2)

    x = x_ref[...]
    gate = jax.lax.dot_general(
        x, w13g_ref[0], (((1,), (1,)), ((), ())),
        preferred_element_type=jnp.float32)
    up = jax.lax.dot_general(
        x, w13u_ref[0], (((1,), (1,)), ((), ())),
        preferred_element_type=jnp.float32)
    act = (gate * jax.nn.sigmoid(gate) * up).astype(jnp.bfloat16)
    partial = jax.lax.dot_general(
        act, w2_ref[0], (((1,), (1,)), ((), ())),
        preferred_element_type=jnp.float32)

    col = _routing_cols(logits_ref[...].astype(jnp.float32), e)
    contrib = partial * col

    @pl.when((e == 0) & (f == 0))
    def _():
        out_ref[...] = contrib

    @pl.when((e > 0) | (f > 0))
    def _():
        out_ref[...] += contrib


def kernel(hidden_states, router_logits, w13_weight, w2_weight):
    x16 = hidden_states.astype(jnp.bfloat16)
    w13_16 = w13_weight.astype(jnp.bfloat16)
    w2_16 = w2_weight.astype(jnp.bfloat16)

    grid = (T // TB, NUM_EXPERTS, NFT)

    out = pl.pallas_call(
        _moe_body,
        grid=grid,
        in_specs=[
            pl.BlockSpec((TB, HIDDEN), lambda t, e, f: (t, 0)),
            pl.BlockSpec((TB, NUM_EXPERTS), lambda t, e, f: (t, 0)),
            pl.BlockSpec((1, FT, HIDDEN), lambda t, e, f: (e, f, 0)),
            pl.BlockSpec((1, FT, HIDDEN), lambda t, e, f: (e, NFT + f, 0)),
            pl.BlockSpec((1, HIDDEN, FT), lambda t, e, f: (e, 0, f)),
        ],
        out_specs=pl.BlockSpec((TB, HIDDEN), lambda t, e, f: (t, 0)),
        out_shape=jax.ShapeDtypeStruct((T, HIDDEN), jnp.float32),
        compiler_params=pltpu.CompilerParams(
            dimension_semantics=("parallel", "arbitrary", "arbitrary"),
        ),
    )(x16, router_logits, w13_16, w13_16, w2_16)
    return out
